# Initial kernel scaffold; baseline (speedup 1.0000x reference)
import jax
import jax.numpy as jnp
from jax.experimental import pallas as pl


def kernel(x, edge_index, We1, be1, We2, be2, Wn1, bn1, Wn2, bn2):
    raise NotImplementedError("write your pallas kernel here")


# speedup vs baseline: 8.8768x; 8.8768x over previous
import jax
import jax.numpy as jnp
import numpy as np
from jax import lax
from jax.experimental import pallas as pl
from jax.experimental.pallas import tpu as pltpu
from jax.experimental.pallas import tpu_sc as plsc

N = 10000
E = 160000
C = 5
F = 68
EF = 64
CF = C * F
CE = C * EF
PADW = 384
DLANE = 368
MW = 352
NW = 32
NPT = 320
EPW = E // NW
EB2 = 40
NB2 = EPW // EB2
BN1 = 1000
BE3 = 2000
CS = 1000
CPG = 4
GR = E // (CS * CPG)


def _ab_tc(x_ref, wt_ref, wb_ref, b1_ref, a_ref, b_ref):
    xb = x_ref[...]
    a = jnp.dot(xb, wt_ref[...], preferred_element_type=jnp.float32) + b1_ref[...]
    i = pl.program_id(0)
    rid = (i * BN1 + lax.broadcasted_iota(jnp.int32, a.shape, 0)).astype(jnp.float32)
    lane = lax.broadcasted_iota(jnp.int32, a.shape, 1)
    a_ref[...] = jnp.where(lane == DLANE, rid, a)
    b_ref[...] = jnp.dot(xb, wb_ref[...], preferred_element_type=jnp.float32)


def _cnt_tc(dst_ref, offs_ref, cnt_ref):
    i = pl.program_id(0)

    @pl.when(i == 0)
    def _():
        cnt_ref[...] = jnp.zeros_like(cnt_ref)

    ones = jnp.ones((CS, 1), jnp.float32)
    for c in range(CPG):
        d = dst_ref[pl.ds(c * CS, CS), :]
        t = jnp.floor((d + 0.5) * (1.0 / NPT))
        oh = (t == lax.broadcasted_iota(jnp.int32, (CS, NW), 1).astype(jnp.float32)).astype(jnp.float32)
        cnt_ref[...] = cnt_ref[...] + lax.dot_general(
            oh, ones, (((0,), (0,)), ((), ())),
            preferred_element_type=jnp.float32)

    @pl.when(i == GR - 1)
    def _():
        ii = lax.broadcasted_iota(jnp.int32, (NW, 64), 0)
        jj = lax.broadcasted_iota(jnp.int32, (NW, 64), 1)
        su = (ii < jj).astype(jnp.float32)
        offs_ref[...] = lax.dot_general(
            cnt_ref[...], su, (((0,), (0,)), ((), ())),
            preferred_element_type=jnp.float32)


def _pos_tc(dst_ref, tril_ref, off_ref, pos_ref, pref_ref):
    i = pl.program_id(0)

    @pl.when(i == 0)
    def _():
        pref_ref[...] = off_ref[...]

    ones = jnp.ones((CS, 1), jnp.float32)
    for c in range(CPG):
        d = dst_ref[pl.ds(c * CS, CS), :]
        t = jnp.floor((d + 0.5) * (1.0 / NPT))
        oh = (t == lax.broadcasted_iota(jnp.int32, (CS, NW), 1).astype(jnp.float32)).astype(jnp.float32)
        b = jnp.dot(tril_ref[...], oh, preferred_element_type=jnp.float32)
        rank1 = jnp.sum(b * oh, axis=1, keepdims=True)
        pstart = jnp.dot(oh, pref_ref[...], preferred_element_type=jnp.float32)
        pos_ref[pl.ds(c * CS, CS), :] = (pstart + rank1).astype(jnp.int32)
        pref_ref[...] = pref_ref[...] + lax.dot_general(
            oh, ones, (((0,), (0,)), ((), ())),
            preferred_element_type=jnp.float32)


def _edge_tc(z_ref, xj_ref, w2_ref, be2_ref, m_ref, msg_ref):
    z = z_ref[...]
    t = jnp.tanh(z)
    logits = jnp.dot(t, w2_ref[...], preferred_element_type=jnp.float32) + be2_ref[...]
    lane = lax.broadcasted_iota(jnp.int32, logits.shape, 1)
    logits = jnp.where(lane < C, logits, -1e30)
    mx = jnp.max(logits, axis=1, keepdims=True)
    p = jnp.exp(logits - mx)
    p = jnp.where(lane < C, p, 0.0)
    w = p / jnp.sum(p, axis=1, keepdims=True)
    wexp = jnp.dot(w, m_ref[...], preferred_element_type=jnp.float32)
    msg = wexp * xj_ref[...]
    lane2 = lax.broadcasted_iota(jnp.int32, msg.shape, 1)
    msg_ref[...] = jnp.where(lane2 == DLANE, z, msg)


def _node_tc(x_ref, g_ref, w1x_ref, w1a_ref, b1_ref, w2_ref, b2_ref, o_ref):
    h = jnp.tanh(
        jnp.dot(x_ref[...], w1x_ref[...], preferred_element_type=jnp.float32)
        + jnp.dot(g_ref[:, :CF], w1a_ref[...], preferred_element_type=jnp.float32)
        + b1_ref[...])
    o_ref[...] = jnp.tanh(
        jnp.dot(h, w2_ref[...], preferred_element_type=jnp.float32) + b2_ref[...])


def _gather_body(a_hbm, b_hbm, xp_hbm, dst_hbm, src_hbm, pos_hbm, z_hbm, xj_hbm,
                 dst_v, src_v, pos_v, zbuf, xbuf, sem_z, sem_x):
    cid = lax.axis_index("c")
    sid = lax.axis_index("s")
    wid = sid * 2 + cid
    base = wid * EPW
    pltpu.sync_copy(dst_hbm.at[pl.ds(base, EPW)], dst_v)
    pltpu.sync_copy(src_hbm.at[pl.ds(base, EPW)], src_v)
    pltpu.sync_copy(pos_hbm.at[pl.ds(base, EPW)], pos_v)

    def block(b, carry):
        pltpu.async_copy(b_hbm.at[src_v.at[pl.ds(b * EB2, EB2)]], zbuf, sem_z).wait()
        pltpu.async_copy(a_hbm.at[dst_v.at[pl.ds(b * EB2, EB2)]], zbuf, sem_z,
                         add=True).wait()
        pltpu.async_copy(xp_hbm.at[src_v.at[pl.ds(b * EB2, EB2)]], xbuf, sem_x).wait()
        pltpu.async_copy(zbuf, z_hbm.at[pos_v.at[pl.ds(b * EB2, EB2)]], sem_z).wait()
        pltpu.async_copy(xbuf, xj_hbm.at[pos_v.at[pl.ds(b * EB2, EB2)]], sem_x).wait()
        return carry

    lax.fori_loop(0, NB2, block, 0)


def _agg_body(msg_hbm, offs_hbm, g_hbm, offv, mbuf, acc, sem):
    cid = lax.axis_index("c")
    sid = lax.axis_index("s")
    t = sid * 2 + cid
    pltpu.sync_copy(offs_hbm, offv)
    ov0 = offv[pl.ds(0, 16)]
    ov1 = offv[pl.ds(16, 16)]
    ov2 = offv[pl.ds(32, 16)]
    lo = jnp.int32(0)
    hi = jnp.int32(0)
    for k in range(32):
        vk = ov0[k] if k < 16 else ov1[k - 16]
        lo = lo + jnp.where(t == k, vk, 0)
    for k in range(1, 33):
        vk = ov0[k] if k < 16 else (ov1[k - 16] if k < 32 else ov2[0])
        hi = hi + jnp.where(t + 1 == k, vk, 0)
    a0 = lo - (lo & 7)
    nblk = (hi - a0 + 7) // 8
    nb = t * NPT

    def zrow(r, c):
        for j in range(MW // 16):
            acc[r, pl.ds(j * 16, 16)] = jnp.zeros((16,), jnp.float32)
        return c

    lax.fori_loop(0, NPT + 1, zrow, 0)

    def blk(j, c):
        r0 = a0 + j * 8
        pltpu.async_copy(msg_hbm.at[pl.ds(pl.multiple_of(r0, 8), 8)], mbuf,
                         sem).wait()
        for k in range(8):
            dv = mbuf[k, pl.ds(DLANE, 16)].astype(jnp.int32)
            loc = dv[0] - nb
            ridx = r0 + k
            bad = (loc < 0) | (loc >= NPT) | (ridx < lo) | (ridx >= hi)
            locc = jnp.where(bad, NPT, loc)
            for j2 in range(MW // 16):
                acc[locc, pl.ds(j2 * 16, 16)] = (
                    acc[locc, pl.ds(j2 * 16, 16)] + mbuf[k, pl.ds(j2 * 16, 16)])
        return c

    lax.fori_loop(0, nblk, blk, 0)

    @pl.when(t < 31)
    def _():
        pltpu.sync_copy(acc.at[pl.ds(0, NPT)], g_hbm.at[pl.ds(t * NPT, NPT)])

    @pl.when(t == 31)
    def _():
        pltpu.sync_copy(acc.at[pl.ds(0, N - 31 * NPT)],
                        g_hbm.at[pl.ds(31 * NPT, N - 31 * NPT)])


def kernel(x, edge_index, We1, be1, We2, be2, Wn1, bn1, Wn2, bn2):
    f32 = jnp.float32
    x2d = x.reshape(N, CF)
    xp = jnp.pad(x2d, ((0, 0), (0, PADW - CF)))
    src32 = edge_index[0].astype(jnp.int32)
    dst32 = edge_index[1].astype(jnp.int32)
    dstf = dst32.astype(f32).reshape(E, 1)

    blkd = jax.scipy.linalg.block_diag
    wt = jnp.pad(blkd(*[We1[c, :F, :] for c in range(C)]), ((0, 0), (0, PADW - CE)))
    wb = jnp.pad(blkd(*[We1[c, F:, :] for c in range(C)]), ((0, 0), (0, PADW - CE)))
    w1x = blkd(*[Wn1[c, :F, :] for c in range(C)])
    w1a = blkd(*[Wn1[c, F:, :] for c in range(C)])
    w2n = blkd(*[Wn2[c] for c in range(C)])
    be1f = jnp.pad(be1.reshape(1, CE), ((0, 0), (0, PADW - CE)))
    bn1f = bn1.reshape(1, CE)
    bn2f = bn2.reshape(1, CE)
    w2e = jnp.zeros((PADW, 128), f32)
    for c in range(C):
        w2e = w2e.at[c * EF:(c + 1) * EF, c].set(We2[c, :, 0])
    be2row = jnp.zeros((1, 128), f32).at[0, :C].set(be2[:, 0])
    cmap = np.zeros((128, PADW), np.float32)
    for l in range(CF):
        cmap[l // F, l] = 1.0
    mexp = jnp.asarray(cmap)
    tril = jnp.asarray(np.tril(np.ones((CS, CS), np.float32), -1))

    a2d, b2d = pl.pallas_call(
        _ab_tc,
        grid=(N // BN1,),
        in_specs=[
            pl.BlockSpec((BN1, CF), lambda i: (i, 0)),
            pl.BlockSpec((CF, PADW), lambda i: (0, 0)),
            pl.BlockSpec((CF, PADW), lambda i: (0, 0)),
            pl.BlockSpec((1, PADW), lambda i: (0, 0)),
        ],
        out_specs=[
            pl.BlockSpec((BN1, PADW), lambda i: (i, 0)),
            pl.BlockSpec((BN1, PADW), lambda i: (i, 0)),
        ],
        out_shape=[
            jax.ShapeDtypeStruct((N, PADW), f32),
            jax.ShapeDtypeStruct((N, PADW), f32),
        ],
    )(x2d, wt, wb, be1f)

    offsf = pl.pallas_call(
        _cnt_tc,
        grid=(GR,),
        in_specs=[pl.BlockSpec((CS * CPG, 1), lambda i: (i, 0))],
        out_specs=pl.BlockSpec((1, 64), lambda i: (0, 0)),
        out_shape=jax.ShapeDtypeStruct((1, 64), f32),
        scratch_shapes=[pltpu.VMEM((NW, 1), f32)],
    )(dstf)
    offs64 = offsf.reshape(64).astype(jnp.int32)
    offcol = offsf[0, :NW].reshape(NW, 1)

    pos = pl.pallas_call(
        _pos_tc,
        grid=(GR,),
        in_specs=[
            pl.BlockSpec((CS * CPG, 1), lambda i: (i, 0)),
            pl.BlockSpec((CS, CS), lambda i: (0, 0)),
            pl.BlockSpec((NW, 1), lambda i: (0, 0)),
        ],
        out_specs=pl.BlockSpec((CS * CPG, 1), lambda i: (i, 0)),
        out_shape=jax.ShapeDtypeStruct((E, 1), jnp.int32),
        scratch_shapes=[pltpu.VMEM((NW, 1), f32)],
    )(dstf, tril, offcol)
    pos1d = pos.reshape(E)

    mesh = plsc.VectorSubcoreMesh(core_axis_name="c", subcore_axis_name="s")

    z_e, xj_e = pl.kernel(
        _gather_body,
        out_type=[
            jax.ShapeDtypeStruct((E, PADW), f32),
            jax.ShapeDtypeStruct((E, PADW), f32),
        ],
        mesh=mesh,
        scratch_types=[
            pltpu.VMEM((EPW,), jnp.int32),
            pltpu.VMEM((EPW,), jnp.int32),
            pltpu.VMEM((EPW,), jnp.int32),
            pltpu.VMEM((EB2, PADW), f32),
            pltpu.VMEM((EB2, PADW), f32),
            pltpu.SemaphoreType.DMA,
            pltpu.SemaphoreType.DMA,
        ],
    )(a2d, b2d, xp, dst32, src32, pos1d)

    msg = pl.pallas_call(
        _edge_tc,
        grid=(E // BE3,),
        in_specs=[
            pl.BlockSpec((BE3, PADW), lambda i: (i, 0)),
            pl.BlockSpec((BE3, PADW), lambda i: (i, 0)),
            pl.BlockSpec((PADW, 128), lambda i: (0, 0)),
            pl.BlockSpec((1, 128), lambda i: (0, 0)),
            pl.BlockSpec((128, PADW), lambda i: (0, 0)),
        ],
        out_specs=pl.BlockSpec((BE3, PADW), lambda i: (i, 0)),
        out_shape=jax.ShapeDtypeStruct((E, PADW), f32),
    )(z_e, xj_e, w2e, be2row, mexp)

    aggr = pl.kernel(
        _agg_body,
        out_type=jax.ShapeDtypeStruct((N, MW), f32),
        mesh=mesh,
        scratch_types=[
            pltpu.VMEM((64,), jnp.int32),
            pltpu.VMEM((8, PADW), f32),
            pltpu.VMEM((NPT + 1, MW), f32),
            pltpu.SemaphoreType.DMA,
        ],
    )(msg, offs64)

    out2d = pl.pallas_call(
        _node_tc,
        grid=(N // BN1,),
        in_specs=[
            pl.BlockSpec((BN1, CF), lambda i: (i, 0)),
            pl.BlockSpec((BN1, MW), lambda i: (i, 0)),
            pl.BlockSpec((CF, CE), lambda i: (0, 0)),
            pl.BlockSpec((CF, CE), lambda i: (0, 0)),
            pl.BlockSpec((1, CE), lambda i: (0, 0)),
            pl.BlockSpec((CE, CE), lambda i: (0, 0)),
            pl.BlockSpec((1, CE), lambda i: (0, 0)),
        ],
        out_specs=pl.BlockSpec((BN1, CE), lambda i: (i, 0)),
        out_shape=jax.ShapeDtypeStruct((N, CE), f32),
    )(x2d, aggr, w1x, w1a, bn1f, w2n, bn2f)

    return out2d.reshape(N, C, EF)
